# Initial kernel scaffold; baseline (speedup 1.0000x reference)
import jax
import jax.numpy as jnp
from jax.experimental import pallas as pl


def kernel(s_j, v_j, r_ij, nbrs, W1, b1, W2, b2, Wd, bd):
    raise NotImplementedError("write your pallas kernel here")


# speedup vs baseline: 15.4627x; 15.4627x over previous
import functools
import math

import jax
import jax.numpy as jnp
import numpy as np
from jax import lax
from jax.experimental import pallas as pl
from jax.experimental.pallas import tpu as pltpu
from jax.experimental.pallas import tpu_sc as plsc

EPS = 1e-15
N_RBF = 20
CUTOFF = 5.0
FEAT = 128
N_NODES = 10000
N_EDGES = 160000

NB_PAD = 32
EDGE_BLK = 1000
GW = 64
SW = 80
N_SUBCORES = 16
N_CORES = 2
TBL = 4 * FEAT
N_PAD = 10240

_vector_mesh = plsc.VectorSubcoreMesh(
    core_axis_name="core", subcore_axis_name="subcore")


@functools.partial(
    pl.kernel,
    out_type=jax.ShapeDtypeStruct((N_EDGES, TBL), jnp.float32),
    mesh=_vector_mesh,
)
def _sc_gather(table_hbm, idx_hbm, o_hbm):
    def body(i_vmem, o_vmem):
        pltpu.sync_copy(table_hbm.at[i_vmem.at[0]], o_vmem)

    pltpu.emit_pipeline(
        body,
        grid=(N_EDGES // GW,),
        in_specs=[pl.BlockSpec((1, GW), lambda i: (i, 0))],
        out_specs=[pl.BlockSpec((GW, TBL), lambda i: (i, 0))],
        core_axis_name=("core", "subcore"),
        dimension_semantics=(pltpu.PARALLEL,),
    )(idx_hbm, o_hbm)


def _mlp_body(g_ref, r4_ref, w1_ref, b1_ref, w2_ref, b2_ref, wd_ref, bd_ref,
              o_ref):
    g = g_ref[...]
    se = g[:, :FEAT]
    h = se @ w1_ref[...] + b1_ref[0:1, :]
    h = h * (1.0 / (1.0 + jnp.exp(-h)))
    phi = h @ w2_ref[...] + b2_ref[0:1, :]

    r4 = r4_ref[...]
    d2 = jnp.sum(r4 * r4, axis=1, keepdims=True) + 3.0 * EPS
    d = jnp.sqrt(d2)
    inv_d = 1.0 / d
    coli = lax.broadcasted_iota(jnp.int32, (EDGE_BLK, NB_PAD), 1)
    col = coli.astype(jnp.float32)
    arg = (col + 1.0) * (math.pi / CUTOFF) * d
    rbf = jnp.where(coli < N_RBF, jnp.sin(arg) * inv_d, 0.0)
    ws = rbf @ wd_ref[...] + bd_ref[0:1, :]

    out = phi * ws
    s0 = out[:, 0:FEAT]
    s1 = out[:, FEAT:2 * FEAT]
    s2 = out[:, 2 * FEAT:3 * FEAT]

    o_ref[:, 0:FEAT] = s1
    for c in range(3):
        u_c = r4[:, c:c + 1] * inv_d
        v_ce = g[:, FEAT * (c + 1):FEAT * (c + 2)]
        o_ref[:, FEAT * (c + 1):FEAT * (c + 2)] = s0 * v_ce + s2 * u_c


_mlp = pl.pallas_call(
    _mlp_body,
    grid=(N_EDGES // EDGE_BLK,),
    in_specs=[
        pl.BlockSpec((EDGE_BLK, TBL), lambda i: (i, 0)),
        pl.BlockSpec((EDGE_BLK, 4), lambda i: (i, 0)),
        pl.BlockSpec((FEAT, FEAT), lambda i: (0, 0)),
        pl.BlockSpec((8, FEAT), lambda i: (0, 0)),
        pl.BlockSpec((FEAT, 3 * FEAT), lambda i: (0, 0)),
        pl.BlockSpec((8, 3 * FEAT), lambda i: (0, 0)),
        pl.BlockSpec((NB_PAD, 3 * FEAT), lambda i: (0, 0)),
        pl.BlockSpec((8, 3 * FEAT), lambda i: (0, 0)),
    ],
    out_specs=pl.BlockSpec((EDGE_BLK, TBL), lambda i: (i, 0)),
    out_shape=jax.ShapeDtypeStruct((N_EDGES, TBL), jnp.float32),
)


def _make_scatter(phase):
    @functools.partial(
        pl.kernel,
        out_type=(jax.ShapeDtypeStruct((N_PAD, FEAT), jnp.float32),
                  jax.ShapeDtypeStruct((N_PAD, FEAT), jnp.float32)),
        mesh=_vector_mesh,
        scratch_types=[
            pltpu.VMEM_SHARED((N_PAD, FEAT), jnp.float32),
            pltpu.VMEM((SW, FEAT), jnp.float32),
            pltpu.VMEM((1, SW), jnp.int32),
        ],
    )
    def sk(delta_hbm, dst_hbm, zeros_hbm, o0_hbm, o1_hbm, acc, dbuf, ibuf):
        core = lax.axis_index("core")
        sub = lax.axis_index("subcore")
        rows = N_PAD // N_SUBCORES
        rbase = sub * rows
        nwin = N_EDGES // (N_SUBCORES * SW)

        def work(plane, o_hbm):
            col0 = plane * FEAT
            pltpu.sync_copy(zeros_hbm.at[pl.ds(rbase, rows)],
                            acc.at[pl.ds(rbase, rows)])
            plsc.subcore_barrier()

            @pl.loop(0, nwin)
            def _(w):
                wi = sub * nwin + w
                ebase = wi * SW
                pltpu.sync_copy(dst_hbm.at[pl.ds(wi, 1)], ibuf)
                pltpu.sync_copy(
                    delta_hbm.at[pl.ds(ebase, SW), pl.ds(col0, FEAT)], dbuf)
                pltpu.sync_copy(dbuf, acc.at[ibuf.at[0]], add=True)

            plsc.subcore_barrier()
            pltpu.sync_copy(acc.at[pl.ds(rbase, rows)],
                            o_hbm.at[pl.ds(rbase, rows)])

        @pl.when(core == 0)
        def _():
            work(2 * phase, o0_hbm)

        @pl.when(core == 1)
        def _():
            work(2 * phase + 1, o1_hbm)

    return sk


_scatter0 = _make_scatter(0)
_scatter1 = _make_scatter(1)


_PERM = np.concatenate([np.arange(FEAT) * 3,
                        np.arange(FEAT) * 3 + 1,
                        np.arange(FEAT) * 3 + 2])


def kernel(s_j, v_j, r_ij, nbrs, W1, b1, W2, b2, Wd, bd):
    table = jnp.concatenate(
        [s_j, v_j[:, :, 0], v_j[:, :, 1], v_j[:, :, 2]], axis=1)
    src2d = nbrs[:, 1].astype(jnp.int32).reshape(N_EDGES // GW, GW)
    dst2d = nbrs[:, 0].astype(jnp.int32).reshape(N_EDGES // SW, SW)
    r4 = jnp.pad(r_ij, ((0, 0), (0, 1)))

    w2p = W2[:, _PERM]
    b2p = jnp.broadcast_to(b2[_PERM].reshape(1, -1), (8, 3 * FEAT))
    wdp = jnp.concatenate(
        [Wd[:, _PERM],
         jnp.zeros((NB_PAD - N_RBF, 3 * FEAT), jnp.float32)], axis=0)
    bdp = jnp.broadcast_to(bd[_PERM].reshape(1, -1), (8, 3 * FEAT))
    b1b = jnp.broadcast_to(b1.reshape(1, -1), (8, FEAT))

    zeros = jnp.zeros((N_PAD, FEAT), jnp.float32)

    g = _sc_gather(table, src2d)
    delta = _mlp(g, r4, W1, b1b, w2p, b2p, wdp, bdp)
    ds_out, dvx = _scatter0(delta, dst2d, zeros)
    dvy, dvz = _scatter1(delta, dst2d, zeros)

    delta_v = jnp.stack(
        [dvx[:N_NODES], dvy[:N_NODES], dvz[:N_NODES]], axis=-1)
    return ds_out[:N_NODES], delta_v


# speedup vs baseline: 18.2709x; 1.1816x over previous
import functools
import math

import jax
import jax.numpy as jnp
import numpy as np
from jax import lax
from jax.experimental import pallas as pl
from jax.experimental.pallas import tpu as pltpu
from jax.experimental.pallas import tpu_sc as plsc

EPS = 1e-15
N_RBF = 20
CUTOFF = 5.0
FEAT = 128
N_NODES = 10000
N_EDGES = 160000

NB_PAD = 32
EDGE_BLK = 1000
GW = 40
SW = 80
N_SUBCORES = 16
N_CORES = 2
N_WORKERS = N_CORES * N_SUBCORES
TBL = 4 * FEAT
N_PAD = 10240

GWIN_PER_W = N_EDGES // (N_WORKERS * GW)
SWIN_PER_S = N_EDGES // (N_SUBCORES * SW)

_vector_mesh = plsc.VectorSubcoreMesh(
    core_axis_name="core", subcore_axis_name="subcore")


def _start(src, dst, sem, add=False):
    pltpu.make_async_copy(src, dst, sem).start(add=add)


def _wait(src, dst, sem):
    pltpu.make_async_copy(src, dst, sem).wait()


@functools.partial(
    pl.kernel,
    out_type=jax.ShapeDtypeStruct((N_EDGES, TBL), jnp.float32),
    mesh=_vector_mesh,
    scratch_types=[
        pltpu.VMEM((GWIN_PER_W, GW), jnp.int32),
        pltpu.VMEM((GW, TBL), jnp.float32),
        pltpu.VMEM((GW, TBL), jnp.float32),
        pltpu.SemaphoreType.DMA,
        pltpu.SemaphoreType.DMA,
        pltpu.SemaphoreType.DMA,
        pltpu.SemaphoreType.DMA,
        pltpu.SemaphoreType.DMA,
    ],
)
def _sc_gather(table_hbm, idx_hbm, o_hbm, iall, bufa, bufb,
               sem_i, sem_ga, sem_gb, sem_oa, sem_ob):
    core = lax.axis_index("core")
    sub = lax.axis_index("subcore")
    wid = sub * N_CORES + core
    lo = wid * GWIN_PER_W

    _start(idx_hbm.at[wid], iall, sem_i)
    _wait(idx_hbm.at[wid], iall, sem_i)

    def g_start(w, buf, sem):
        _start(table_hbm.at[iall.at[w]], buf, sem)

    def g_wait(buf, sem):
        _wait(table_hbm.at[iall.at[0]], buf, sem)

    def o_slice(w):
        return o_hbm.at[pl.ds((lo + w) * GW, GW), :]

    g_start(0, bufa, sem_ga)
    g_start(1, bufb, sem_gb)

    @pl.loop(0, (GWIN_PER_W - 1) // 2)
    def _(p):
        w0 = 2 * p
        g_wait(bufa, sem_ga)
        _start(bufa, o_slice(w0), sem_oa)
        g_wait(bufb, sem_gb)
        _start(bufb, o_slice(w0 + 1), sem_ob)
        _wait(bufa, o_slice(w0), sem_oa)
        g_start(w0 + 2, bufa, sem_ga)
        _wait(bufb, o_slice(w0 + 1), sem_ob)

        @pl.when(p < (GWIN_PER_W - 1) // 2 - 1)
        def _():
            g_start(w0 + 3, bufb, sem_gb)

    wlast = GWIN_PER_W - 1
    g_wait(bufa, sem_ga)
    _start(bufa, o_slice(wlast), sem_oa)
    _wait(bufa, o_slice(wlast), sem_oa)


def _mlp_body(g_ref, r4_ref, w1_ref, b1_ref, w2_ref, b2_ref, wd_ref, bd_ref,
              os_ref, o0_ref, o1_ref, o2_ref):
    g = g_ref[...]
    se = g[:, :FEAT]
    h = se @ w1_ref[...] + b1_ref[0:1, :]
    h = h * (1.0 / (1.0 + jnp.exp(-h)))
    phi = h @ w2_ref[...] + b2_ref[0:1, :]

    r4 = r4_ref[...]
    d2 = jnp.sum(r4 * r4, axis=1, keepdims=True) + 3.0 * EPS
    d = jnp.sqrt(d2)
    inv_d = 1.0 / d
    coli = lax.broadcasted_iota(jnp.int32, (EDGE_BLK, NB_PAD), 1)
    col = coli.astype(jnp.float32)
    arg = (col + 1.0) * (math.pi / CUTOFF) * d
    rbf = jnp.where(coli < N_RBF, jnp.sin(arg) * inv_d, 0.0)
    ws = rbf @ wd_ref[...] + bd_ref[0:1, :]

    out = phi * ws
    s0 = out[:, 0:FEAT]
    s1 = out[:, FEAT:2 * FEAT]
    s2 = out[:, 2 * FEAT:3 * FEAT]

    os_ref[...] = s1
    for c, o_ref in enumerate((o0_ref, o1_ref, o2_ref)):
        u_c = r4[:, c:c + 1] * inv_d
        v_ce = g[:, FEAT * (c + 1):FEAT * (c + 2)]
        o_ref[...] = s0 * v_ce + s2 * u_c


_plane = jax.ShapeDtypeStruct((N_EDGES, FEAT), jnp.float32)
_blk128 = pl.BlockSpec((EDGE_BLK, FEAT), lambda i: (i, 0))
_mlp = pl.pallas_call(
    _mlp_body,
    grid=(N_EDGES // EDGE_BLK,),
    in_specs=[
        pl.BlockSpec((EDGE_BLK, TBL), lambda i: (i, 0)),
        pl.BlockSpec((EDGE_BLK, 4), lambda i: (i, 0)),
        pl.BlockSpec((FEAT, FEAT), lambda i: (0, 0)),
        pl.BlockSpec((8, FEAT), lambda i: (0, 0)),
        pl.BlockSpec((FEAT, 3 * FEAT), lambda i: (0, 0)),
        pl.BlockSpec((8, 3 * FEAT), lambda i: (0, 0)),
        pl.BlockSpec((NB_PAD, 3 * FEAT), lambda i: (0, 0)),
        pl.BlockSpec((8, 3 * FEAT), lambda i: (0, 0)),
    ],
    out_specs=(_blk128, _blk128, _blk128, _blk128),
    out_shape=(_plane, _plane, _plane, _plane),
)


@functools.partial(
    pl.kernel,
    out_type=(jax.ShapeDtypeStruct((N_PAD, FEAT), jnp.float32),
              jax.ShapeDtypeStruct((N_PAD, FEAT), jnp.float32)),
    mesh=_vector_mesh,
    scratch_types=[
        pltpu.VMEM_SHARED((N_PAD, FEAT), jnp.float32),
        pltpu.VMEM((SWIN_PER_S, SW), jnp.int32),
        pltpu.VMEM((SW, FEAT), jnp.float32),
        pltpu.VMEM((SW, FEAT), jnp.float32),
        pltpu.SemaphoreType.DMA,
        pltpu.SemaphoreType.DMA,
        pltpu.SemaphoreType.DMA,
        pltpu.SemaphoreType.DMA,
        pltpu.SemaphoreType.DMA,
    ],
)
def _sc_scatter(pa_hbm, pb_hbm, dst_hbm, zeros_hbm, o0_hbm, o1_hbm,
                acc, iall, da, db, sem_i, sem_a, sem_b, sem_sa, sem_sb):
    core = lax.axis_index("core")
    sub = lax.axis_index("subcore")
    rows = N_PAD // N_SUBCORES
    rbase = sub * rows

    _start(dst_hbm.at[sub], iall, sem_i)

    def work(p_hbm, o_hbm):
        def in_slice(w):
            return p_hbm.at[pl.ds((sub * SWIN_PER_S + w) * SW, SW), :]

        def sc_start(w, buf, sem):
            _start(buf, acc.at[iall.at[w]], sem, add=True)

        def sc_wait(buf, sem):
            _wait(buf, acc.at[iall.at[0]], sem)

        _start(in_slice(0), da, sem_a)
        _start(in_slice(1), db, sem_b)

        pltpu.sync_copy(zeros_hbm.at[pl.ds(rbase, rows)],
                        acc.at[pl.ds(rbase, rows)])
        _wait(dst_hbm.at[sub], iall, sem_i)
        plsc.subcore_barrier()

        @pl.loop(0, (SWIN_PER_S - 1) // 2)
        def _(p):
            w0 = 2 * p
            _wait(in_slice(w0), da, sem_a)
            sc_start(w0, da, sem_sa)
            _wait(in_slice(w0 + 1), db, sem_b)
            sc_start(w0 + 1, db, sem_sb)
            sc_wait(da, sem_sa)
            _start(in_slice(w0 + 2), da, sem_a)
            sc_wait(db, sem_sb)

            @pl.when(p < (SWIN_PER_S - 1) // 2 - 1)
            def _():
                _start(in_slice(w0 + 3), db, sem_b)

        wlast = SWIN_PER_S - 1
        _wait(in_slice(wlast), da, sem_a)
        pltpu.sync_copy(da, acc.at[iall.at[wlast]], add=True)

        plsc.subcore_barrier()
        pltpu.sync_copy(acc.at[pl.ds(rbase, rows)],
                        o_hbm.at[pl.ds(rbase, rows)])

    @pl.when(core == 0)
    def _():
        work(pa_hbm, o0_hbm)

    @pl.when(core == 1)
    def _():
        work(pb_hbm, o1_hbm)


_PERM = np.concatenate([np.arange(FEAT) * 3,
                        np.arange(FEAT) * 3 + 1,
                        np.arange(FEAT) * 3 + 2])


def kernel(s_j, v_j, r_ij, nbrs, W1, b1, W2, b2, Wd, bd):
    table = jnp.concatenate(
        [s_j, v_j[:, :, 0], v_j[:, :, 1], v_j[:, :, 2]], axis=1)
    src3d = nbrs[:, 1].astype(jnp.int32).reshape(N_WORKERS, GWIN_PER_W, GW)
    dst3d = nbrs[:, 0].astype(jnp.int32).reshape(N_SUBCORES, SWIN_PER_S, SW)
    r4 = jnp.pad(r_ij, ((0, 0), (0, 1)))

    w2p = W2[:, _PERM]
    b2p = jnp.broadcast_to(b2[_PERM].reshape(1, -1), (8, 3 * FEAT))
    wdp = jnp.concatenate(
        [Wd[:, _PERM],
         jnp.zeros((NB_PAD - N_RBF, 3 * FEAT), jnp.float32)], axis=0)
    bdp = jnp.broadcast_to(bd[_PERM].reshape(1, -1), (8, 3 * FEAT))
    b1b = jnp.broadcast_to(b1.reshape(1, -1), (8, FEAT))

    zeros = jnp.zeros((N_PAD, FEAT), jnp.float32)

    g = _sc_gather(table, src3d)
    ds_p, dv0_p, dv1_p, dv2_p = _mlp(g, r4, W1, b1b, w2p, b2p, wdp, bdp)
    ds_out, dvx = _sc_scatter(ds_p, dv0_p, dst3d, zeros)
    dvy, dvz = _sc_scatter(dv1_p, dv2_p, dst3d, zeros)

    delta_v = jnp.stack(
        [dvx[:N_NODES], dvy[:N_NODES], dvz[:N_NODES]], axis=-1)
    return ds_out[:N_NODES], delta_v


# speedup vs baseline: 20.5244x; 1.1233x over previous
import functools
import math

import jax
import jax.numpy as jnp
import numpy as np
from jax import lax
from jax.experimental import pallas as pl
from jax.experimental.pallas import tpu as pltpu
from jax.experimental.pallas import tpu_sc as plsc

EPS = 1e-15
N_RBF = 20
CUTOFF = 5.0
FEAT = 128
N_NODES = 10000
N_EDGES = 160000

NB_PAD = 24
EDGE_BLK = 1280
GW = 40
SW = 80
SWD = 40
N_SUBCORES = 16
N_CORES = 2
N_WORKERS = N_CORES * N_SUBCORES
TBL = 4 * FEAT
DVC = 192
N_PAD = 10240

GWIN_PER_W = N_EDGES // (N_WORKERS * GW)
SWIN_PER_S = N_EDGES // (N_SUBCORES * SW)
DWIN_PER_W = N_EDGES // (N_WORKERS * SWD)

_vector_mesh = plsc.VectorSubcoreMesh(
    core_axis_name="core", subcore_axis_name="subcore")


def _start(src, dst, sem, add=False):
    pltpu.make_async_copy(src, dst, sem).start(add=add)


def _wait(src, dst, sem):
    pltpu.make_async_copy(src, dst, sem).wait()


@functools.partial(
    pl.kernel,
    out_type=jax.ShapeDtypeStruct((N_EDGES, TBL), jnp.float32),
    mesh=_vector_mesh,
    scratch_types=[
        pltpu.VMEM((GWIN_PER_W, GW), jnp.int32),
        pltpu.VMEM((GW, TBL), jnp.float32),
        pltpu.VMEM((GW, TBL), jnp.float32),
        pltpu.SemaphoreType.DMA,
        pltpu.SemaphoreType.DMA,
        pltpu.SemaphoreType.DMA,
        pltpu.SemaphoreType.DMA,
        pltpu.SemaphoreType.DMA,
    ],
)
def _sc_gather(table_hbm, idx_hbm, o_hbm, iall, bufa, bufb,
               sem_i, sem_ga, sem_gb, sem_oa, sem_ob):
    core = lax.axis_index("core")
    sub = lax.axis_index("subcore")
    wid = sub * N_CORES + core
    lo = wid * GWIN_PER_W

    _start(idx_hbm.at[wid], iall, sem_i)
    _wait(idx_hbm.at[wid], iall, sem_i)

    def g_start(w, buf, sem):
        _start(table_hbm.at[iall.at[w]], buf, sem)

    def g_wait(buf, sem):
        _wait(table_hbm.at[iall.at[0]], buf, sem)

    def o_slice(w):
        return o_hbm.at[pl.ds((lo + w) * GW, GW), :]

    g_start(0, bufa, sem_ga)
    g_start(1, bufb, sem_gb)

    @pl.loop(0, (GWIN_PER_W - 1) // 2)
    def _(p):
        w0 = 2 * p
        g_wait(bufa, sem_ga)
        _start(bufa, o_slice(w0), sem_oa)
        g_wait(bufb, sem_gb)
        _start(bufb, o_slice(w0 + 1), sem_ob)
        _wait(bufa, o_slice(w0), sem_oa)
        g_start(w0 + 2, bufa, sem_ga)
        _wait(bufb, o_slice(w0 + 1), sem_ob)

        @pl.when(p < (GWIN_PER_W - 1) // 2 - 1)
        def _():
            g_start(w0 + 3, bufb, sem_gb)

    wlast = GWIN_PER_W - 1
    g_wait(bufa, sem_ga)
    _start(bufa, o_slice(wlast), sem_oa)
    _wait(bufa, o_slice(wlast), sem_oa)


def _mlp_body(g_ref, rt_ref, w1_ref, b1_ref, w2_ref, b2_ref, wd_ref, bd_ref,
              rx_ref, tu_ref, os_ref, oa_ref, ob_ref, oc_ref):
    g = g_ref[...]
    se = g[:, :FEAT]
    h = se @ w1_ref[...] + b1_ref[0:1, :]
    h = h * (1.0 / (1.0 + jnp.exp(-h)))
    phi = h @ w2_ref[...] + b2_ref[0:1, :]

    rt = rt_ref[...]
    x_ = rt[0:1, :]
    y_ = rt[1:2, :]
    z_ = rt[2:3, :]
    d2t = x_ * x_ + y_ * y_ + z_ * z_ + 3.0 * EPS
    dt = jnp.sqrt(d2t)
    inv_dt = 1.0 / dt
    th = (math.pi / CUTOFF) * dt
    s1 = jnp.sin(th) * inv_dt
    c2 = 2.0 * jnp.cos(th)
    rows = [s1]
    prev2 = jnp.zeros_like(s1)
    prev1 = s1
    for _ in range(N_RBF - 1):
        cur = c2 * prev1 - prev2
        rows.append(cur)
        prev2, prev1 = prev1, cur
    for _ in range(NB_PAD - N_RBF):
        rows.append(jnp.zeros_like(s1))
    rbf = jnp.concatenate(rows, axis=0).T
    ws = rbf @ wd_ref[...] + bd_ref[0:1, :]

    out = phi * ws
    s0 = out[:, 0:FEAT]
    s1o = out[:, FEAT:2 * FEAT]
    s2 = out[:, 2 * FEAT:3 * FEAT]

    os_ref[...] = s1o

    u8 = jnp.concatenate(
        [x_ * inv_dt, y_ * inv_dt, z_ * inv_dt] + [jnp.zeros_like(s1)] * 5,
        axis=0).T
    s0x = s0 @ rx_ref[...]
    s2x = s2 @ rx_ref[...]
    ut = u8 @ tu_ref[...]
    dv = s0x * g[:, FEAT:] + s2x * ut
    oa_ref[...] = dv[:, 0:FEAT]
    ob_ref[...] = dv[:, FEAT:2 * FEAT]
    oc_ref[...] = dv[:, 2 * FEAT:]


_mlp = pl.pallas_call(
    _mlp_body,
    grid=(N_EDGES // EDGE_BLK,),
    in_specs=[
        pl.BlockSpec((EDGE_BLK, TBL), lambda i: (i, 0)),
        pl.BlockSpec((8, EDGE_BLK), lambda i: (0, i)),
        pl.BlockSpec((FEAT, FEAT), lambda i: (0, 0)),
        pl.BlockSpec((8, FEAT), lambda i: (0, 0)),
        pl.BlockSpec((FEAT, 3 * FEAT), lambda i: (0, 0)),
        pl.BlockSpec((8, 3 * FEAT), lambda i: (0, 0)),
        pl.BlockSpec((NB_PAD, 3 * FEAT), lambda i: (0, 0)),
        pl.BlockSpec((8, 3 * FEAT), lambda i: (0, 0)),
        pl.BlockSpec((FEAT, 3 * FEAT), lambda i: (0, 0)),
        pl.BlockSpec((8, 3 * FEAT), lambda i: (0, 0)),
    ],
    out_specs=tuple(
        pl.BlockSpec((EDGE_BLK, FEAT), lambda i: (i, 0)) for _ in range(4)),
    out_shape=tuple(
        jax.ShapeDtypeStruct((N_EDGES, FEAT), jnp.float32) for _ in range(4)),
)


def _scatter_loop(in_slice, iall, acc, da, db, sem_a, sem_b, sem_sa, sem_sb,
                  nwin):
    def sc_start(w, buf, sem):
        _start(buf, acc.at[iall.at[w]], sem, add=True)

    def sc_wait(buf, sem):
        _wait(buf, acc.at[iall.at[0]], sem)

    @pl.loop(0, (nwin - 1) // 2)
    def _(p):
        w0 = 2 * p
        _wait(in_slice(w0), da, sem_a)
        sc_start(w0, da, sem_sa)
        _wait(in_slice(w0 + 1), db, sem_b)
        sc_start(w0 + 1, db, sem_sb)
        sc_wait(da, sem_sa)
        _start(in_slice(w0 + 2), da, sem_a)
        sc_wait(db, sem_sb)

        @pl.when(p < (nwin - 1) // 2 - 1)
        def _():
            _start(in_slice(w0 + 3), db, sem_b)

    wlast = nwin - 1
    _wait(in_slice(wlast), da, sem_a)
    pltpu.sync_copy(da, acc.at[iall.at[wlast]], add=True)


@functools.partial(
    pl.kernel,
    out_type=(jax.ShapeDtypeStruct((N_PAD, FEAT), jnp.float32),
              jax.ShapeDtypeStruct((N_PAD, FEAT), jnp.float32)),
    mesh=_vector_mesh,
    scratch_types=[
        pltpu.VMEM_SHARED((N_PAD, FEAT), jnp.float32),
        pltpu.VMEM((SWIN_PER_S, SW), jnp.int32),
        pltpu.VMEM((SW, FEAT), jnp.float32),
        pltpu.VMEM((SW, FEAT), jnp.float32),
        pltpu.SemaphoreType.DMA,
        pltpu.SemaphoreType.DMA,
        pltpu.SemaphoreType.DMA,
        pltpu.SemaphoreType.DMA,
        pltpu.SemaphoreType.DMA,
    ],
)
def _sc_scatter(pa_hbm, pb_hbm, dst_hbm, zeros_hbm, o0_hbm, o1_hbm,
                acc, iall, da, db, sem_i, sem_a, sem_b, sem_sa, sem_sb):
    core = lax.axis_index("core")
    sub = lax.axis_index("subcore")
    rows = N_PAD // N_SUBCORES
    rbase = sub * rows

    _start(dst_hbm.at[sub], iall, sem_i)

    def work(p_hbm, o_hbm):
        def in_slice(w):
            return p_hbm.at[pl.ds((sub * SWIN_PER_S + w) * SW, SW), :]

        _start(in_slice(0), da, sem_a)
        _start(in_slice(1), db, sem_b)
        pltpu.sync_copy(zeros_hbm.at[pl.ds(rbase, rows)],
                        acc.at[pl.ds(rbase, rows)])
        _wait(dst_hbm.at[sub], iall, sem_i)
        plsc.subcore_barrier()
        _scatter_loop(in_slice, iall, acc, da, db,
                      sem_a, sem_b, sem_sa, sem_sb, SWIN_PER_S)
        plsc.subcore_barrier()
        pltpu.sync_copy(acc.at[pl.ds(rbase, rows)],
                        o_hbm.at[pl.ds(rbase, rows)])

    @pl.when(core == 0)
    def _():
        work(pa_hbm, o0_hbm)

    @pl.when(core == 1)
    def _():
        work(pb_hbm, o1_hbm)


_PERM = np.concatenate([np.arange(FEAT) * 3,
                        np.arange(FEAT) * 3 + 1,
                        np.arange(FEAT) * 3 + 2])

_RX = np.zeros((FEAT, 3 * FEAT), np.float32)
_RX[np.repeat(np.arange(FEAT), 3), np.arange(3 * FEAT)] = 1.0
_TU = np.zeros((8, 3 * FEAT), np.float32)
_TU[np.tile(np.arange(3), FEAT), np.arange(3 * FEAT)] = 1.0


def kernel(s_j, v_j, r_ij, nbrs, W1, b1, W2, b2, Wd, bd):
    table = jnp.concatenate([s_j, v_j.reshape(N_NODES, 3 * FEAT)], axis=1)
    src3d = nbrs[:, 1].astype(jnp.int32).reshape(N_WORKERS, GWIN_PER_W, GW)
    dst3d = nbrs[:, 0].astype(jnp.int32).reshape(N_SUBCORES, SWIN_PER_S, SW)
    rt8 = jnp.concatenate(
        [r_ij.T, jnp.zeros((5, N_EDGES), jnp.float32)], axis=0)

    w2p = W2[:, _PERM]
    b2p = jnp.broadcast_to(b2[_PERM].reshape(1, -1), (8, 3 * FEAT))
    wdp = jnp.concatenate(
        [Wd[:, _PERM],
         jnp.zeros((NB_PAD - N_RBF, 3 * FEAT), jnp.float32)], axis=0)
    bdp = jnp.broadcast_to(bd[_PERM].reshape(1, -1), (8, 3 * FEAT))
    b1b = jnp.broadcast_to(b1.reshape(1, -1), (8, FEAT))

    zeros = jnp.zeros((N_PAD, FEAT), jnp.float32)

    g = _sc_gather(table, src3d)
    ds_p, dva_p, dvb_p, dvc_p = _mlp(g, rt8, W1, b1b, w2p, b2p, wdp, bdp,
                                     jnp.asarray(_RX), jnp.asarray(_TU))
    q0, q1 = _sc_scatter(dva_p, dvb_p, dst3d, zeros)
    q2, ds_out = _sc_scatter(dvc_p, ds_p, dst3d, zeros)

    dv_out = jnp.concatenate(
        [q0[:N_NODES], q1[:N_NODES], q2[:N_NODES]],
        axis=1).reshape(N_NODES, FEAT, 3)
    return ds_out[:N_NODES], dv_out
